# baseline (speedup 1.0000x reference)
import functools

import jax
import jax.numpy as jnp
from jax import lax
from jax.experimental import pallas as pl
from jax.experimental.pallas import tpu as pltpu
from jax.experimental.pallas import tpu_sc as plsc

B = 16384
EMB = 32
HIST = 50
FC_OUT = 64
NC = 2
NS = 16
NW = NC * NS
BPW = B // NW
CH = 128
NCH = BPW // CH


def _relayout_body(x_ref, eye_ref, o_ref):
    x = x_ref[...]
    br = o_ref.shape[0]
    eye = eye_ref[...]
    dn = (((0,), (0,)), ((), ()))
    o_ref[...] = jnp.concatenate(
        [lax.dot_general(x[:, k * br:(k + 1) * br], eye, dn,
                         preferred_element_type=jnp.float32)
         for k in range(4)], axis=1)


RELAY_BR = 2048


def _relayout(table):
    br = RELAY_BR
    n = table.shape[0]
    grid = -(-n // (4 * br))
    np_ = grid * br
    packed = pl.pallas_call(
        _relayout_body,
        grid=(grid,),
        in_specs=[pl.BlockSpec((32, 4 * br), lambda i: (0, i)),
                  pl.BlockSpec((EMB, EMB), lambda i: (0, 0))],
        out_specs=pl.BlockSpec((br, 128), lambda i: (i, 0)),
        out_shape=jax.ShapeDtypeStruct((np_, 128), jnp.float32),
    )(table.T, jnp.eye(EMB, dtype=jnp.float32))
    return packed.reshape(4 * np_, EMB)


def _permute_idx(m):
    br = RELAY_BR
    g = 4 * br
    blk = m // g
    c = m % g
    return (blk * br + c % br) * 4 + c // br


def _sc_gather(item_p, brand_p, desc_p, scat_p, item_table, brand_table, desc_table):
    mesh = plsc.VectorSubcoreMesh(core_axis_name="c", subcore_axis_name="s")

    @functools.partial(
        pl.kernel,
        out_type=jax.ShapeDtypeStruct((3, B, EMB), jnp.float32),
        mesh=mesh,
        scratch_types=[
            pltpu.VMEM((NCH, CH), jnp.int32),
            pltpu.VMEM((NCH, CH), jnp.int32),
            pltpu.VMEM((HIST, NCH, CH), jnp.int32),
            pltpu.VMEM((NCH, CH), jnp.int32),
            pltpu.VMEM((BPW, EMB), jnp.float32),
            pltpu.VMEM((BPW, EMB), jnp.float32),
            pltpu.VMEM((NCH, CH, EMB), jnp.float32),
            pltpu.VMEM_SHARED((NS * BPW, EMB), jnp.float32),
            pltpu.SemaphoreType.DMA,
            pltpu.SemaphoreType.DMA,
            pltpu.SemaphoreType.DMA,
            pltpu.SemaphoreType.DMA,
            pltpu.SemaphoreType.DMA,
        ],
        compiler_params=pltpu.CompilerParams(use_tc_tiling_on_sc=False),
    )
    def k(item_hbm, brand_hbm, desc_hbm, scat_hbm, itab, btab, dtab, out_hbm,
          item_v, brand_v, desc_v, scat_v, item_rows, brand_rows, ring, acc_sh,
          sem_ib, sem0, sem1, sem2, sem3):
        c = lax.axis_index("c")
        s = lax.axis_index("s")
        wid = s * NC + c
        base = wid * BPW
        sems = (sem0, sem1, sem2, sem3)

        pltpu.sync_copy(item_hbm.at[wid], item_v)
        pltpu.sync_copy(brand_hbm.at[wid], brand_v)
        pltpu.sync_copy(scat_hbm.at[s], scat_v)
        pltpu.sync_copy(desc_hbm.at[:, wid], desc_v)

        for u in range(NCH):
            pltpu.async_copy(dtab.at[desc_v.at[0, u]], ring.at[u], sems[u])
        for u in range(NCH):
            pltpu.async_copy(itab.at[item_v.at[u]],
                             item_rows.at[pl.ds(u * CH, CH)], sem_ib)
            pltpu.async_copy(btab.at[brand_v.at[u]],
                             brand_rows.at[pl.ds(u * CH, CH)], sem_ib)

        def body(j, _):
            for u in range(NCH):
                pltpu.make_async_copy(dtab.at[desc_v.at[j, u]], ring.at[u],
                                      sems[u]).wait()
                dst = acc_sh.at[scat_v.at[u]]
                @pl.when(j == 0)
                def _():
                    pltpu.sync_copy(ring.at[u], dst)
                @pl.when(j > 0)
                def _():
                    pltpu.sync_copy(ring.at[u], dst, add=True)
                @pl.when(j < HIST - 1)
                def _():
                    pltpu.async_copy(dtab.at[desc_v.at[j + 1, u]], ring.at[u],
                                     sems[u])
            return 0

        lax.fori_loop(0, HIST, body, 0)

        for u in range(NCH):
            pltpu.make_async_copy(itab.at[item_v.at[u]],
                                  item_rows.at[pl.ds(u * CH, CH)], sem_ib).wait()
            pltpu.make_async_copy(btab.at[brand_v.at[u]],
                                  brand_rows.at[pl.ds(u * CH, CH)], sem_ib).wait()
        pltpu.sync_copy(item_rows, out_hbm.at[0, pl.ds(base, BPW)])
        pltpu.sync_copy(brand_rows, out_hbm.at[1, pl.ds(base, BPW)])
        pltpu.sync_copy(acc_sh.at[pl.ds(s * BPW, BPW)],
                        out_hbm.at[2, pl.ds(base, BPW)])

    return k(item_p, brand_p, desc_p, scat_p, item_table, brand_table, desc_table)


def _fc_body(feat_ref, price_ref, w0, w1, w2, wp, b2, out_ref):
    acc = jnp.dot(feat_ref[0], w0[...], preferred_element_type=jnp.float32)
    acc = acc + jnp.dot(feat_ref[1], w1[...], preferred_element_type=jnp.float32)
    acc = acc + jnp.dot(feat_ref[2], w2[...], preferred_element_type=jnp.float32)
    acc = acc + price_ref[...] * wp[...]
    out_ref[...] = acc + b2[...]


def kernel(item, brand, price, description, item_table, brand_table, desc_table, W, b):
    item_p = _permute_idx(item.astype(jnp.int32)).reshape(NW, NCH, CH)
    brand_p = brand.astype(jnp.int32).reshape(NW, NCH, CH)
    desc_p = (_permute_idx(description.astype(jnp.int32))
              .T.reshape(HIST, NW, NCH, CH))
    scat_p = (jnp.arange(NS, dtype=jnp.int32)[:, None, None] * BPW
              + jnp.arange(NCH, dtype=jnp.int32)[None, :, None] * CH
              + jnp.arange(CH, dtype=jnp.int32)[None, None, :])

    itab = _relayout(item_table)
    dtab = _relayout(desc_table)
    feat = _sc_gather(item_p, brand_p, desc_p, scat_p,
                      itab, brand_table, dtab)

    W0 = W[0:EMB]
    W1 = W[EMB:2 * EMB]
    W2 = W[2 * EMB:3 * EMB] * jnp.float32(1.0 / HIST)
    wp = W[3 * EMB:3 * EMB + 1]
    b2 = b.reshape(1, FC_OUT)
    price2 = price.reshape(B, 1)

    BK = 2048
    out = pl.pallas_call(
        _fc_body,
        grid=(B // BK,),
        in_specs=[
            pl.BlockSpec((3, BK, EMB), lambda i: (0, i, 0)),
            pl.BlockSpec((BK, 1), lambda i: (i, 0)),
            pl.BlockSpec((EMB, FC_OUT), lambda i: (0, 0)),
            pl.BlockSpec((EMB, FC_OUT), lambda i: (0, 0)),
            pl.BlockSpec((EMB, FC_OUT), lambda i: (0, 0)),
            pl.BlockSpec((1, FC_OUT), lambda i: (0, 0)),
            pl.BlockSpec((1, FC_OUT), lambda i: (0, 0)),
        ],
        out_specs=pl.BlockSpec((BK, FC_OUT), lambda i: (i, 0)),
        out_shape=jax.ShapeDtypeStruct((B, FC_OUT), jnp.float32),
    )(feat, price2, W0, W1, W2, wp, b2)
    return out


# speedup vs baseline: 1.0182x; 1.0182x over previous
import functools

import jax
import jax.numpy as jnp
from jax import lax
from jax.experimental import pallas as pl
from jax.experimental.pallas import tpu as pltpu
from jax.experimental.pallas import tpu_sc as plsc

B = 16384
EMB = 32
HIST = 50
FC_OUT = 64
NC = 2
NS = 16
NW = NC * NS
BPW = B // NW
CH = 128
NCH = BPW // CH


def _relayout_body(x_ref, o_ref):
    x = x_ref[...]
    br = o_ref.shape[0]
    o_ref[...] = jnp.concatenate(
        [x[:, k * br:(k + 1) * br].T for k in range(4)], axis=1)


RELAY_BR = 4096


def _relayout(table):
    br = RELAY_BR
    n = table.shape[0]
    grid = -(-n // (4 * br))
    np_ = grid * br
    packed = pl.pallas_call(
        _relayout_body,
        grid=(grid,),
        in_specs=[pl.BlockSpec((32, 4 * br), lambda i: (0, i))],
        out_specs=pl.BlockSpec((br, 128), lambda i: (i, 0)),
        out_shape=jax.ShapeDtypeStruct((np_, 128), jnp.float32),
    )(table.T)
    return packed.reshape(4 * np_, EMB)


def _permute_idx(m):
    br = RELAY_BR
    g = 4 * br
    blk = m // g
    c = m % g
    return (blk * br + c % br) * 4 + c // br


def _sc_desc_brand(brand_p, desc_p, scat_p, brand_table, dtab):
    mesh = plsc.VectorSubcoreMesh(core_axis_name="c", subcore_axis_name="s")

    @functools.partial(
        pl.kernel,
        out_type=jax.ShapeDtypeStruct((2, B, EMB), jnp.float32),
        mesh=mesh,
        scratch_types=[
            pltpu.VMEM((NCH, CH), jnp.int32),
            pltpu.VMEM((HIST, NCH, CH), jnp.int32),
            pltpu.VMEM((NCH, CH), jnp.int32),
            pltpu.VMEM((BPW, EMB), jnp.float32),
            pltpu.VMEM((NCH, CH, EMB), jnp.float32),
            pltpu.VMEM_SHARED((NS * BPW, EMB), jnp.float32),
            pltpu.SemaphoreType.DMA,
            pltpu.SemaphoreType.DMA,
            pltpu.SemaphoreType.DMA,
            pltpu.SemaphoreType.DMA,
            pltpu.SemaphoreType.DMA,
        ],
        compiler_params=pltpu.CompilerParams(use_tc_tiling_on_sc=False),
    )
    def k(brand_hbm, desc_hbm, scat_hbm, btab, dtab_hbm, out_hbm,
          brand_v, desc_v, scat_v, brand_rows, ring, acc_sh,
          sem_b, sem0, sem1, sem2, sem3):
        c = lax.axis_index("c")
        s = lax.axis_index("s")
        wid = s * NC + c
        base = wid * BPW
        sems = (sem0, sem1, sem2, sem3)

        pltpu.sync_copy(brand_hbm.at[wid], brand_v)
        pltpu.sync_copy(scat_hbm.at[s], scat_v)
        pltpu.sync_copy(desc_hbm.at[:, wid], desc_v)

        for u in range(NCH):
            pltpu.async_copy(dtab_hbm.at[desc_v.at[0, u]], ring.at[u], sems[u])
        for u in range(NCH):
            pltpu.async_copy(btab.at[brand_v.at[u]],
                             brand_rows.at[pl.ds(u * CH, CH)], sem_b)

        def body(j, _):
            for u in range(NCH):
                pltpu.make_async_copy(dtab_hbm.at[desc_v.at[j, u]], ring.at[u],
                                      sems[u]).wait()
                dst = acc_sh.at[scat_v.at[u]]
                @pl.when(j == 0)
                def _():
                    pltpu.sync_copy(ring.at[u], dst)
                @pl.when(j > 0)
                def _():
                    pltpu.sync_copy(ring.at[u], dst, add=True)
                @pl.when(j < HIST - 1)
                def _():
                    pltpu.async_copy(dtab_hbm.at[desc_v.at[j + 1, u]],
                                     ring.at[u], sems[u])
            return 0

        lax.fori_loop(0, HIST, body, 0)

        for u in range(NCH):
            pltpu.make_async_copy(btab.at[brand_v.at[u]],
                                  brand_rows.at[pl.ds(u * CH, CH)], sem_b).wait()
        pltpu.sync_copy(brand_rows, out_hbm.at[0, pl.ds(base, BPW)])
        pltpu.sync_copy(acc_sh.at[pl.ds(s * BPW, BPW)],
                        out_hbm.at[1, pl.ds(base, BPW)])

    return k(brand_p, desc_p, scat_p, brand_table, dtab)


def _sc_item(item_p, itab):
    mesh = plsc.VectorSubcoreMesh(core_axis_name="c", subcore_axis_name="s")

    @functools.partial(
        pl.kernel,
        out_type=jax.ShapeDtypeStruct((B, EMB), jnp.float32),
        mesh=mesh,
        scratch_types=[
            pltpu.VMEM((NCH, CH), jnp.int32),
            pltpu.VMEM((BPW, EMB), jnp.float32),
            pltpu.SemaphoreType.DMA,
        ],
        compiler_params=pltpu.CompilerParams(use_tc_tiling_on_sc=False),
    )
    def k(item_hbm, itab_hbm, out_hbm, item_v, rows, sem):
        c = lax.axis_index("c")
        s = lax.axis_index("s")
        wid = s * NC + c
        base = wid * BPW
        pltpu.sync_copy(item_hbm.at[wid], item_v)
        for u in range(NCH):
            pltpu.async_copy(itab_hbm.at[item_v.at[u]],
                             rows.at[pl.ds(u * CH, CH)], sem)
        for u in range(NCH):
            pltpu.make_async_copy(itab_hbm.at[item_v.at[u]],
                                  rows.at[pl.ds(u * CH, CH)], sem).wait()
        pltpu.sync_copy(rows, out_hbm.at[pl.ds(base, BPW)])

    return k(item_p, itab)


def _fc_body(itemf_ref, db_ref, price_ref, w0, w1, w2, wp, b2, out_ref):
    acc = jnp.dot(itemf_ref[...], w0[...], preferred_element_type=jnp.float32)
    acc = acc + jnp.dot(db_ref[0], w1[...], preferred_element_type=jnp.float32)
    acc = acc + jnp.dot(db_ref[1], w2[...], preferred_element_type=jnp.float32)
    acc = acc + price_ref[...] * wp[...]
    out_ref[...] = acc + b2[...]


def kernel(item, brand, price, description, item_table, brand_table, desc_table, W, b):
    item_p = _permute_idx(item.astype(jnp.int32)).reshape(NW, NCH, CH)
    brand_p = brand.astype(jnp.int32).reshape(NW, NCH, CH)
    desc_p = (_permute_idx(description.astype(jnp.int32))
              .T.reshape(HIST, NW, NCH, CH))
    scat_p = (jnp.arange(NS, dtype=jnp.int32)[:, None, None] * BPW
              + jnp.arange(NCH, dtype=jnp.int32)[None, :, None] * CH
              + jnp.arange(CH, dtype=jnp.int32)[None, None, :])

    dtab = _relayout(desc_table)
    feat_db = _sc_desc_brand(brand_p, desc_p, scat_p, brand_table, dtab)
    itab = _relayout(item_table)
    item_f = _sc_item(item_p, itab)

    W0 = W[0:EMB]
    W1 = W[EMB:2 * EMB]
    W2 = W[2 * EMB:3 * EMB] * jnp.float32(1.0 / HIST)
    wp = W[3 * EMB:3 * EMB + 1]
    b2 = b.reshape(1, FC_OUT)
    price2 = price.reshape(B, 1)

    BK = 2048
    out = pl.pallas_call(
        _fc_body,
        grid=(B // BK,),
        in_specs=[
            pl.BlockSpec((BK, EMB), lambda i: (i, 0)),
            pl.BlockSpec((2, BK, EMB), lambda i: (0, i, 0)),
            pl.BlockSpec((BK, 1), lambda i: (i, 0)),
            pl.BlockSpec((EMB, FC_OUT), lambda i: (0, 0)),
            pl.BlockSpec((EMB, FC_OUT), lambda i: (0, 0)),
            pl.BlockSpec((EMB, FC_OUT), lambda i: (0, 0)),
            pl.BlockSpec((1, FC_OUT), lambda i: (0, 0)),
            pl.BlockSpec((1, FC_OUT), lambda i: (0, 0)),
        ],
        out_specs=pl.BlockSpec((BK, FC_OUT), lambda i: (i, 0)),
        out_shape=jax.ShapeDtypeStruct((B, FC_OUT), jnp.float32),
    )(item_f, feat_db, price2, W0, W1, W2, wp, b2)
    return out


# speedup vs baseline: 1.0205x; 1.0022x over previous
import functools

import jax
import jax.numpy as jnp
from jax import lax
from jax.experimental import pallas as pl
from jax.experimental.pallas import tpu as pltpu
from jax.experimental.pallas import tpu_sc as plsc

B = 16384
EMB = 32
HIST = 50
FC_OUT = 64
NC = 2
NS = 16
NW = NC * NS
BPW = B // NW
CH = 128
NCH = BPW // CH


def _relayout_body(x_ref, o_ref):
    x = x_ref[...]
    br = o_ref.shape[0]
    o_ref[...] = jnp.concatenate(
        [x[:, k * br:(k + 1) * br].T for k in range(4)], axis=1)


RELAY_BR = 4096


def _relayout(table):
    br = RELAY_BR
    n = table.shape[0]
    grid = -(-n // (4 * br))
    np_ = grid * br
    packed = pl.pallas_call(
        _relayout_body,
        grid=(grid,),
        in_specs=[pl.BlockSpec((32, 4 * br), lambda i: (0, i))],
        out_specs=pl.BlockSpec((br, 128), lambda i: (i, 0)),
        out_shape=jax.ShapeDtypeStruct((np_, 128), jnp.float32),
        compiler_params=pltpu.CompilerParams(
            dimension_semantics=("parallel",)),
    )(table.T)
    return packed.reshape(4 * np_, EMB)


def _permute_idx(m):
    br = RELAY_BR
    g = 4 * br
    blk = m // g
    c = m % g
    return (blk * br + c % br) * 4 + c // br


def _sc_desc_brand(brand_p, desc_p, scat_p, brand_table, dtab):
    mesh = plsc.VectorSubcoreMesh(core_axis_name="c", subcore_axis_name="s")

    @functools.partial(
        pl.kernel,
        out_type=jax.ShapeDtypeStruct((2, B, EMB), jnp.float32),
        mesh=mesh,
        scratch_types=[
            pltpu.VMEM((NCH, CH), jnp.int32),
            pltpu.VMEM((HIST, NCH, CH), jnp.int32),
            pltpu.VMEM((NCH, CH), jnp.int32),
            pltpu.VMEM((BPW, EMB), jnp.float32),
            pltpu.VMEM((NCH, CH, EMB), jnp.float32),
            pltpu.VMEM_SHARED((NS * BPW, EMB), jnp.float32),
            pltpu.SemaphoreType.DMA,
            pltpu.SemaphoreType.DMA,
            pltpu.SemaphoreType.DMA,
            pltpu.SemaphoreType.DMA,
            pltpu.SemaphoreType.DMA,
        ],
        compiler_params=pltpu.CompilerParams(use_tc_tiling_on_sc=False),
    )
    def k(brand_hbm, desc_hbm, scat_hbm, btab, dtab_hbm, out_hbm,
          brand_v, desc_v, scat_v, brand_rows, ring, acc_sh,
          sem_b, sem0, sem1, sem2, sem3):
        c = lax.axis_index("c")
        s = lax.axis_index("s")
        wid = s * NC + c
        base = wid * BPW
        sems = (sem0, sem1, sem2, sem3)

        pltpu.sync_copy(brand_hbm.at[wid], brand_v)
        pltpu.sync_copy(scat_hbm.at[s], scat_v)
        pltpu.sync_copy(desc_hbm.at[:, wid], desc_v)

        for u in range(NCH):
            pltpu.async_copy(dtab_hbm.at[desc_v.at[0, u]], ring.at[u], sems[u])
        for u in range(NCH):
            pltpu.async_copy(btab.at[brand_v.at[u]],
                             brand_rows.at[pl.ds(u * CH, CH)], sem_b)

        def body(j, _):
            for u in range(NCH):
                pltpu.make_async_copy(dtab_hbm.at[desc_v.at[j, u]], ring.at[u],
                                      sems[u]).wait()
                dst = acc_sh.at[scat_v.at[u]]
                @pl.when(j == 0)
                def _():
                    pltpu.sync_copy(ring.at[u], dst)
                @pl.when(j > 0)
                def _():
                    pltpu.sync_copy(ring.at[u], dst, add=True)
                @pl.when(j < HIST - 1)
                def _():
                    pltpu.async_copy(dtab_hbm.at[desc_v.at[j + 1, u]],
                                     ring.at[u], sems[u])
            return 0

        lax.fori_loop(0, HIST, body, 0)

        for u in range(NCH):
            pltpu.make_async_copy(btab.at[brand_v.at[u]],
                                  brand_rows.at[pl.ds(u * CH, CH)], sem_b).wait()
        pltpu.sync_copy(brand_rows, out_hbm.at[0, pl.ds(base, BPW)])
        pltpu.sync_copy(acc_sh.at[pl.ds(s * BPW, BPW)],
                        out_hbm.at[1, pl.ds(base, BPW)])

    return k(brand_p, desc_p, scat_p, brand_table, dtab)


def _sc_item(item_p, itab):
    mesh = plsc.VectorSubcoreMesh(core_axis_name="c", subcore_axis_name="s")

    @functools.partial(
        pl.kernel,
        out_type=jax.ShapeDtypeStruct((B, EMB), jnp.float32),
        mesh=mesh,
        scratch_types=[
            pltpu.VMEM((NCH, CH), jnp.int32),
            pltpu.VMEM((BPW, EMB), jnp.float32),
            pltpu.SemaphoreType.DMA,
        ],
        compiler_params=pltpu.CompilerParams(use_tc_tiling_on_sc=False),
    )
    def k(item_hbm, itab_hbm, out_hbm, item_v, rows, sem):
        c = lax.axis_index("c")
        s = lax.axis_index("s")
        wid = s * NC + c
        base = wid * BPW
        pltpu.sync_copy(item_hbm.at[wid], item_v)
        for u in range(NCH):
            pltpu.async_copy(itab_hbm.at[item_v.at[u]],
                             rows.at[pl.ds(u * CH, CH)], sem)
        for u in range(NCH):
            pltpu.make_async_copy(itab_hbm.at[item_v.at[u]],
                                  rows.at[pl.ds(u * CH, CH)], sem).wait()
        pltpu.sync_copy(rows, out_hbm.at[pl.ds(base, BPW)])

    return k(item_p, itab)


def _fc_body(itemf_ref, db_ref, price_ref, w0, w1, w2, wp, b2, out_ref):
    acc = jnp.dot(itemf_ref[...], w0[...], preferred_element_type=jnp.float32)
    acc = acc + jnp.dot(db_ref[0], w1[...], preferred_element_type=jnp.float32)
    acc = acc + jnp.dot(db_ref[1], w2[...], preferred_element_type=jnp.float32)
    acc = acc + price_ref[...] * wp[...]
    out_ref[...] = acc + b2[...]


def kernel(item, brand, price, description, item_table, brand_table, desc_table, W, b):
    item_p = _permute_idx(item.astype(jnp.int32)).reshape(NW, NCH, CH)
    brand_p = brand.astype(jnp.int32).reshape(NW, NCH, CH)
    desc_p = (_permute_idx(description.astype(jnp.int32))
              .T.reshape(HIST, NW, NCH, CH))
    scat_p = (jnp.arange(NS, dtype=jnp.int32)[:, None, None] * BPW
              + jnp.arange(NCH, dtype=jnp.int32)[None, :, None] * CH
              + jnp.arange(CH, dtype=jnp.int32)[None, None, :])

    dtab = _relayout(desc_table)
    feat_db = _sc_desc_brand(brand_p, desc_p, scat_p, brand_table, dtab)
    itab = _relayout(item_table)
    item_f = _sc_item(item_p, itab)

    W0 = W[0:EMB]
    W1 = W[EMB:2 * EMB]
    W2 = W[2 * EMB:3 * EMB] * jnp.float32(1.0 / HIST)
    wp = W[3 * EMB:3 * EMB + 1]
    b2 = b.reshape(1, FC_OUT)
    price2 = price.reshape(B, 1)

    BK = 2048
    out = pl.pallas_call(
        _fc_body,
        grid=(B // BK,),
        in_specs=[
            pl.BlockSpec((BK, EMB), lambda i: (i, 0)),
            pl.BlockSpec((2, BK, EMB), lambda i: (0, i, 0)),
            pl.BlockSpec((BK, 1), lambda i: (i, 0)),
            pl.BlockSpec((EMB, FC_OUT), lambda i: (0, 0)),
            pl.BlockSpec((EMB, FC_OUT), lambda i: (0, 0)),
            pl.BlockSpec((EMB, FC_OUT), lambda i: (0, 0)),
            pl.BlockSpec((1, FC_OUT), lambda i: (0, 0)),
            pl.BlockSpec((1, FC_OUT), lambda i: (0, 0)),
        ],
        out_specs=pl.BlockSpec((BK, FC_OUT), lambda i: (i, 0)),
        out_shape=jax.ShapeDtypeStruct((B, FC_OUT), jnp.float32),
    )(item_f, feat_db, price2, W0, W1, W2, wp, b2)
    return out
